# baseline (speedup 1.0000x reference)
import jax
import jax.numpy as jnp
from jax.experimental import pallas as pl
from jax.experimental.pallas import tpu as pltpu

B, T, F = 16, 2048, 1024
H = F // 2


def _body(x0_ref, x1_ref, o_ref):
    b = pl.program_id(0)
    m0 = jnp.max(jnp.abs(x0_ref[0]), axis=1)
    m1 = jnp.max(jnp.abs(x1_ref[0]), axis=1)
    m = jnp.maximum(m0, m1)
    c = jnp.sum((m > 0.0).astype(jnp.int32))
    t = jnp.maximum(c - 1, 0)
    o_ref[pl.ds(b, 1), :H] = x0_ref[0, pl.ds(t, 1), :]
    o_ref[pl.ds(b, 1), H:] = x1_ref[0, pl.ds(t, 1), :]


_fused = pl.pallas_call(
    _body,
    grid=(B,),
    in_specs=[
        pl.BlockSpec((1, T, H), lambda b: (b, 0, 0)),
        pl.BlockSpec((1, T, H), lambda b: (b, 0, 1)),
    ],
    out_specs=pl.BlockSpec((B, F), lambda b: (0, 0)),
    out_shape=jax.ShapeDtypeStruct((B, F), jnp.float32),
    compiler_params=pltpu.CompilerParams(
        dimension_semantics=("arbitrary",),
    ),
)


def kernel(inputs):
    return _fused(inputs, inputs)


# speedup vs baseline: 1.0106x; 1.0106x over previous
import jax
import jax.numpy as jnp
from jax.experimental import pallas as pl
from jax.experimental.pallas import tpu as pltpu

B, T, F = 16, 2048, 1024
G = 2


def _body(x_ref, o_ref):
    g = pl.program_id(0)
    for j in range(G):
        b = g * G + j
        m = jnp.max(jnp.abs(x_ref[j]), axis=1)
        c = jnp.sum((m > 0.0).astype(jnp.int32))
        t = jnp.maximum(c - 1, 0)
        o_ref[pl.ds(b, 1), :] = x_ref[j, pl.ds(t, 1), :]


_fused = pl.pallas_call(
    _body,
    grid=(B // G,),
    in_specs=[pl.BlockSpec((G, T, F), lambda g: (g, 0, 0))],
    out_specs=pl.BlockSpec((B, F), lambda g: (0, 0)),
    out_shape=jax.ShapeDtypeStruct((B, F), jnp.float32),
    compiler_params=pltpu.CompilerParams(
        dimension_semantics=("arbitrary",),
    ),
)


def kernel(inputs):
    return _fused(inputs)
